# baseline (speedup 1.0000x reference)
import functools

import jax
import jax.numpy as jnp
from jax import lax
from jax.experimental import pallas as pl
from jax.experimental.pallas import tpu as pltpu
from jax.experimental.pallas import tpu_sc as plsc

NC = 2
NS = 16
NW = NC * NS
CHUNK = 160
R = 4
LAG = 2


def _make_gather_pipelined(n_pad, d, chunk, r, lag):
    n_w = n_pad // NW
    n_chunks = n_w // chunk
    n_groups = n_chunks // r
    mesh = plsc.VectorSubcoreMesh(core_axis_name="c", subcore_axis_name="s")

    @functools.partial(
        pl.kernel,
        mesh=mesh,
        out_type=jax.ShapeDtypeStruct((n_pad, d), jnp.float32),
        scratch_types=[
            pltpu.VMEM((n_w,), jnp.int32),
            *[pltpu.VMEM((chunk, d), jnp.float32) for _ in range(r)],
            *[pltpu.SemaphoreType.DMA for _ in range(2 * r)],
        ],
    )
    def gather_kernel(table_hbm, idx_hbm, out_hbm, idx_v, *rest):
        rows = rest[:r]
        sg = rest[r:2 * r]
        ss = rest[2 * r:3 * r]
        wid = lax.axis_index("s") * NC + lax.axis_index("c")
        chunk0 = wid * n_chunks

        pltpu.sync_copy(idx_hbm.at[pl.ds(wid * n_w, n_w)], idx_v)

        def fire_gather(i, b):
            src = table_hbm.at[idx_v.at[pl.ds(i * chunk, chunk)]]
            pltpu.async_copy(src, rows[b], sg[b])

        def wait_gather(b):
            src = table_hbm.at[idx_v.at[pl.ds(0, chunk)]]
            pltpu.make_async_copy(src, rows[b], sg[b]).wait()

        def fire_store(i, b):
            dst = out_hbm.at[pl.ds((chunk0 + i) * chunk, chunk)]
            pltpu.async_copy(rows[b], dst, ss[b])

        def wait_store(b):
            dst = out_hbm.at[pl.ds(0, chunk)]
            pltpu.make_async_copy(rows[b], dst, ss[b]).wait()

        for i in range(r):
            fire_gather(i, i)
            if i >= lag:
                wait_gather(i - lag)
                fire_store(i - lag, i - lag)

        def body(g, carry):
            for b in range(r):
                i = g * r + b
                wait_store(b)
                fire_gather(i, b)
                bl = (b + r - lag) % r
                wait_gather(bl)
                fire_store(i - lag, bl)
            return carry

        lax.fori_loop(1, n_groups, body, 0)

        for i in range(n_chunks - lag, n_chunks):
            b = i % r
            wait_gather(b)
            fire_store(i, b)
        for b in range(r):
            wait_store(b)

    return gather_kernel


def _make_gather_simple(n_pad, d, chunk):
    n_w = n_pad // NW
    n_chunks = n_w // chunk
    mesh = plsc.VectorSubcoreMesh(core_axis_name="c", subcore_axis_name="s")

    @functools.partial(
        pl.kernel,
        mesh=mesh,
        out_type=jax.ShapeDtypeStruct((n_pad, d), jnp.float32),
        scratch_types=[
            pltpu.VMEM((chunk,), jnp.int32),
            pltpu.VMEM((chunk, d), jnp.float32),
            pltpu.SemaphoreType.DMA,
        ],
    )
    def gather_kernel(table_hbm, idx_hbm, out_hbm, idx_v, rows_v, sem):
        wid = lax.axis_index("s") * NC + lax.axis_index("c")
        w_base = wid * n_w

        def body(i, carry):
            base = w_base + i * chunk
            pltpu.sync_copy(idx_hbm.at[pl.ds(base, chunk)], idx_v)
            pltpu.async_copy(table_hbm.at[idx_v], rows_v, sem).wait()
            pltpu.sync_copy(rows_v, out_hbm.at[pl.ds(base, chunk)])
            return carry

        lax.fori_loop(0, n_chunks, body, 0)

    return gather_kernel


def kernel(indices, W):
    b, l = indices.shape
    _, d = W.shape
    n = b * l
    idx_flat = indices.reshape(n).astype(jnp.int32)
    grain = NW * CHUNK * R
    n_pad = ((n + grain - 1) // grain) * grain
    if n_pad != n:
        idx_flat = jnp.pad(idx_flat, (0, n_pad - n))
    n_chunks_w = n_pad // NW // CHUNK
    if n_chunks_w // R >= 2 and (n_chunks_w + R * d) * CHUNK * 4 <= 500_000:
        out = _make_gather_pipelined(n_pad, d, CHUNK, R, LAG)(W, idx_flat)
    else:
        out = _make_gather_simple(n_pad, d, 128)(W, idx_flat)
    if n_pad != n:
        out = out[:n]
    return out.reshape(b, l, d)


# speedup vs baseline: 1.0026x; 1.0026x over previous
import functools

import jax
import jax.numpy as jnp
from jax import lax
from jax.experimental import pallas as pl
from jax.experimental.pallas import tpu as pltpu
from jax.experimental.pallas import tpu_sc as plsc

NC = 2
NS = 16
NW = NC * NS
CHUNK = 128
R = 5
LAG = 3


def _make_gather_pipelined(n_pad, d, chunk, r, lag):
    n_w = n_pad // NW
    n_chunks = n_w // chunk
    n_groups = n_chunks // r
    mesh = plsc.VectorSubcoreMesh(core_axis_name="c", subcore_axis_name="s")

    @functools.partial(
        pl.kernel,
        mesh=mesh,
        out_type=jax.ShapeDtypeStruct((n_pad, d), jnp.float32),
        scratch_types=[
            pltpu.VMEM((n_w,), jnp.int32),
            *[pltpu.VMEM((chunk, d), jnp.float32) for _ in range(r)],
            *[pltpu.SemaphoreType.DMA for _ in range(2 * r)],
        ],
    )
    def gather_kernel(table_hbm, idx_hbm, out_hbm, idx_v, *rest):
        rows = rest[:r]
        sg = rest[r:2 * r]
        ss = rest[2 * r:3 * r]
        wid = lax.axis_index("s") * NC + lax.axis_index("c")
        chunk0 = wid * n_chunks

        pltpu.sync_copy(idx_hbm.at[pl.ds(wid * n_w, n_w)], idx_v)

        def fire_gather(i, b):
            src = table_hbm.at[idx_v.at[pl.ds(i * chunk, chunk)]]
            pltpu.async_copy(src, rows[b], sg[b])

        def wait_gather(b):
            src = table_hbm.at[idx_v.at[pl.ds(0, chunk)]]
            pltpu.make_async_copy(src, rows[b], sg[b]).wait()

        def fire_store(i, b):
            dst = out_hbm.at[pl.ds((chunk0 + i) * chunk, chunk)]
            pltpu.async_copy(rows[b], dst, ss[b])

        def wait_store(b):
            dst = out_hbm.at[pl.ds(0, chunk)]
            pltpu.make_async_copy(rows[b], dst, ss[b]).wait()

        for i in range(r):
            fire_gather(i, i)
            if i >= lag:
                wait_gather(i - lag)
                fire_store(i - lag, i - lag)

        def body(g, carry):
            for b in range(r):
                i = g * r + b
                wait_store(b)
                fire_gather(i, b)
                bl = (b + r - lag) % r
                wait_gather(bl)
                fire_store(i - lag, bl)
            return carry

        lax.fori_loop(1, n_groups, body, 0)

        for i in range(n_chunks - lag, n_chunks):
            b = i % r
            wait_gather(b)
            fire_store(i, b)
        for b in range(r):
            wait_store(b)

    return gather_kernel


def _make_gather_simple(n_pad, d, chunk):
    n_w = n_pad // NW
    n_chunks = n_w // chunk
    mesh = plsc.VectorSubcoreMesh(core_axis_name="c", subcore_axis_name="s")

    @functools.partial(
        pl.kernel,
        mesh=mesh,
        out_type=jax.ShapeDtypeStruct((n_pad, d), jnp.float32),
        scratch_types=[
            pltpu.VMEM((chunk,), jnp.int32),
            pltpu.VMEM((chunk, d), jnp.float32),
            pltpu.SemaphoreType.DMA,
        ],
    )
    def gather_kernel(table_hbm, idx_hbm, out_hbm, idx_v, rows_v, sem):
        wid = lax.axis_index("s") * NC + lax.axis_index("c")
        w_base = wid * n_w

        def body(i, carry):
            base = w_base + i * chunk
            pltpu.sync_copy(idx_hbm.at[pl.ds(base, chunk)], idx_v)
            pltpu.async_copy(table_hbm.at[idx_v], rows_v, sem).wait()
            pltpu.sync_copy(rows_v, out_hbm.at[pl.ds(base, chunk)])
            return carry

        lax.fori_loop(0, n_chunks, body, 0)

    return gather_kernel


def kernel(indices, W):
    b, l = indices.shape
    _, d = W.shape
    n = b * l
    idx_flat = indices.reshape(n).astype(jnp.int32)
    grain = NW * CHUNK * R
    n_pad = ((n + grain - 1) // grain) * grain
    if n_pad != n:
        idx_flat = jnp.pad(idx_flat, (0, n_pad - n))
    n_chunks_w = n_pad // NW // CHUNK
    if n_chunks_w // R >= 2 and (n_chunks_w + R * d) * CHUNK * 4 <= 500_000:
        out = _make_gather_pipelined(n_pad, d, CHUNK, R, LAG)(W, idx_flat)
    else:
        out = _make_gather_simple(n_pad, d, 128)(W, idx_flat)
    if n_pad != n:
        out = out[:n]
    return out.reshape(b, l, d)


# speedup vs baseline: 1.0105x; 1.0078x over previous
import functools

import jax
import jax.numpy as jnp
from jax import lax
from jax.experimental import pallas as pl
from jax.experimental.pallas import tpu as pltpu
from jax.experimental.pallas import tpu_sc as plsc

NC = 2
NS = 16
NW = NC * NS
CHUNK = 128
R = 5
LAG = 3


def _make_gather_pipelined(n_pad, d, chunk, r, lag):
    n_w = n_pad // NW
    n_chunks = n_w // chunk
    n_groups = n_chunks // r
    mesh = plsc.VectorSubcoreMesh(core_axis_name="c", subcore_axis_name="s")

    @functools.partial(
        pl.kernel,
        mesh=mesh,
        out_type=jax.ShapeDtypeStruct((n_pad, d), jnp.float32),
        scratch_types=[
            pltpu.VMEM((n_w,), jnp.int32),
            *[pltpu.VMEM((chunk, d), jnp.float32) for _ in range(r)],
            *[pltpu.SemaphoreType.DMA for _ in range(2 * r)],
        ],
    )
    def gather_kernel(table_hbm, idx_hbm, out_hbm, idx_v, *rest):
        rows = rest[:r]
        sg = rest[r:2 * r]
        ss = rest[2 * r:3 * r]
        wid = lax.axis_index("s") * NC + lax.axis_index("c")
        chunk0 = wid * n_chunks

        pltpu.sync_copy(idx_hbm.at[pl.ds(wid * n_w, n_w)], idx_v)

        def fire_gather(i, b):
            src = table_hbm.at[idx_v.at[pl.ds(i * chunk, chunk)]]
            pltpu.async_copy(src, rows[b], sg[b])

        def wait_gather(b):
            src = table_hbm.at[idx_v.at[pl.ds(0, chunk)]]
            pltpu.make_async_copy(src, rows[b], sg[b]).wait()

        def fire_store(i, b):
            dst = out_hbm.at[pl.ds((i * NW + wid) * chunk, chunk)]
            pltpu.async_copy(rows[b], dst, ss[b])

        def wait_store(b):
            dst = out_hbm.at[pl.ds(0, chunk)]
            pltpu.make_async_copy(rows[b], dst, ss[b]).wait()

        for i in range(r):
            fire_gather(i, i)
            if i >= lag:
                wait_gather(i - lag)
                fire_store(i - lag, i - lag)

        def body(g, carry):
            for b in range(r):
                i = g * r + b
                wait_store(b)
                fire_gather(i, b)
                bl = (b + r - lag) % r
                wait_gather(bl)
                fire_store(i - lag, bl)
            return carry

        lax.fori_loop(1, n_groups, body, 0)

        for i in range(n_chunks - lag, n_chunks):
            b = i % r
            wait_gather(b)
            fire_store(i, b)
        for b in range(r):
            wait_store(b)

    return gather_kernel


def _make_gather_simple(n_pad, d, chunk):
    n_w = n_pad // NW
    n_chunks = n_w // chunk
    mesh = plsc.VectorSubcoreMesh(core_axis_name="c", subcore_axis_name="s")

    @functools.partial(
        pl.kernel,
        mesh=mesh,
        out_type=jax.ShapeDtypeStruct((n_pad, d), jnp.float32),
        scratch_types=[
            pltpu.VMEM((chunk,), jnp.int32),
            pltpu.VMEM((chunk, d), jnp.float32),
            pltpu.SemaphoreType.DMA,
        ],
    )
    def gather_kernel(table_hbm, idx_hbm, out_hbm, idx_v, rows_v, sem):
        wid = lax.axis_index("s") * NC + lax.axis_index("c")
        w_base = wid * n_w

        def body(i, carry):
            base = w_base + i * chunk
            pltpu.sync_copy(idx_hbm.at[pl.ds(base, chunk)], idx_v)
            pltpu.async_copy(table_hbm.at[idx_v], rows_v, sem).wait()
            pltpu.sync_copy(rows_v, out_hbm.at[pl.ds(base, chunk)])
            return carry

        lax.fori_loop(0, n_chunks, body, 0)

    return gather_kernel


def kernel(indices, W):
    b, l = indices.shape
    _, d = W.shape
    n = b * l
    idx_flat = indices.reshape(n).astype(jnp.int32)
    grain = NW * CHUNK * R
    n_pad = ((n + grain - 1) // grain) * grain
    if n_pad != n:
        idx_flat = jnp.pad(idx_flat, (0, n_pad - n))
    n_chunks_w = n_pad // NW // CHUNK
    if n_chunks_w // R >= 2 and (n_chunks_w + R * d) * CHUNK * 4 <= 500_000:
        idx_perm = (idx_flat.reshape(n_chunks_w, NW, CHUNK)
                    .transpose(1, 0, 2).reshape(n_pad))
        out = _make_gather_pipelined(n_pad, d, CHUNK, R, LAG)(W, idx_perm)
    else:
        out = _make_gather_simple(n_pad, d, 128)(W, idx_flat)
    if n_pad != n:
        out = out[:n]
    return out.reshape(b, l, d)
